# baseline (speedup 1.0000x reference)
import functools

import jax
import jax.numpy as jnp
from jax.experimental import pallas as pl
import jax.experimental.pallas.tpu as pltpu

_T = 2048
_H = 1024
_I = 2816
_E = 8
_B = 256
_NB = 24


def _router_kernel(x_ref, gw_ref, pos_ref, tw_ref,
                   be_ref, bv_ref, xbf_ref):
    x = x_ref[...]
    xbf_ref[...] = x.astype(jnp.bfloat16)
    gw = gw_ref[...]
    logits = jax.lax.dot_general(
        x, gw, (((1,), (1,)), ((), ())),
        precision=jax.lax.Precision.DEFAULT,
        preferred_element_type=jnp.float32)
    m = jnp.max(logits, axis=1, keepdims=True)
    ex = jnp.exp(logits - m)
    probs = ex / jnp.sum(ex, axis=1, keepdims=True)

    iota_e = jax.lax.broadcasted_iota(jnp.int32, (_T, _E), 1)
    m1 = jnp.max(probs, axis=1, keepdims=True)
    i0 = jnp.min(jnp.where(probs == m1, iota_e, _E), axis=1, keepdims=True)
    a0 = (iota_e == i0)
    probs2 = jnp.where(a0, -1.0, probs)
    m2 = jnp.max(probs2, axis=1, keepdims=True)
    i1 = jnp.min(jnp.where(probs2 == m2, iota_e, _E), axis=1, keepdims=True)
    a1 = (iota_e == i1)

    a0i = a0.astype(jnp.int32)
    a1i = a1.astype(jnp.int32)

    def inclusive_scan(a):
        c = a
        d = 1
        while d < _T:
            c = c + jnp.concatenate(
                [jnp.zeros((d, _E), jnp.int32), c[:-d]], axis=0)
            d *= 2
        return c

    cs = inclusive_scan(a0i + a1i)
    counts = cs[_T - 1:_T, :]
    aligned = jnp.bitwise_and(counts + (_B - 1), ~(_B - 1))

    offs_cols = []
    run = jnp.zeros((1, 1), jnp.int32)
    for e in range(_E):
        offs_cols.append(run)
        run = run + aligned[:, e:e + 1]
    offs = jnp.concatenate(offs_cols, axis=1)
    total = run

    rank0 = jnp.sum(a0i * (cs - 1), axis=1, keepdims=True)
    rank1 = jnp.sum(a1i * (cs - 1), axis=1, keepdims=True)
    off0 = jnp.sum(a0i * offs, axis=1, keepdims=True)
    off1 = jnp.sum(a1i * offs, axis=1, keepdims=True)
    pos_ref[...] = jnp.concatenate([off0 + rank0, off1 + rank1], axis=1)
    tw_ref[...] = jnp.concatenate([m1, m2], axis=1)

    bstart = _B * jax.lax.broadcasted_iota(jnp.int32, (_NB, 1), 0)
    inb = jnp.logical_and(bstart >= offs, bstart < offs + aligned)
    e_row = jax.lax.broadcasted_iota(jnp.int32, (_NB, _E), 1)
    be = jnp.sum(jnp.where(inb, e_row, 0), axis=1, keepdims=True)
    bv = jnp.sum(inb.astype(jnp.int32), axis=1, keepdims=True)
    lastq = total - 1
    in_last = jnp.logical_and(lastq >= offs, lastq < offs + aligned)
    e_last = jnp.sum(jnp.where(
        in_last, jax.lax.broadcasted_iota(jnp.int32, (1, _E), 1), 0),
        axis=1, keepdims=True)
    be_ref[...] = jnp.where(bv > 0, be, e_last)
    bv_ref[...] = bv


def _ffn_kernel(be_ref, bv_ref, pos_ref, tw_ref,
                x_ref, w1_ref, w3_ref, w2_ref, out_ref, xg_all):
    i = pl.program_id(0)
    b = pl.program_id(1)

    @pl.when(jnp.logical_and(b == 0, i == 0))
    def _init():
        out_ref[...] = jnp.zeros_like(out_ref)

    @pl.when(bv_ref[b] > 0)
    def _body():
        slots = _B * b + jax.lax.broadcasted_iota(jnp.int32, (1, _B), 1)
        m0 = (pos_ref[:, 0:1] == slots)
        m1 = (pos_ref[:, 1:2] == slots)
        gt = jnp.logical_or(m0, m1).astype(jnp.bfloat16)
        wsrow = jnp.sum(jnp.where(m0, tw_ref[:, 0:1], 0.0) +
                        jnp.where(m1, tw_ref[:, 1:2], 0.0),
                        axis=0, keepdims=True)

        @pl.when(i == 0)
        def _gather():
            xg_all[:, pl.ds(_B * b, _B)] = jax.lax.dot_general(
                x_ref[...], gt, (((0,), (0,)), ((), ())),
                preferred_element_type=jnp.float32).astype(jnp.bfloat16)

        xg = xg_all[:, pl.ds(_B * b, _B)]
        w1 = w1_ref[0].astype(jnp.bfloat16)
        w3 = w3_ref[0].astype(jnp.bfloat16)
        w2 = w2_ref[0].astype(jnp.bfloat16)
        gp = jax.lax.dot_general(
            w1, xg, (((1,), (0,)), ((), ())),
            preferred_element_type=jnp.float32)
        up = jax.lax.dot_general(
            w3, xg, (((1,), (0,)), ((), ())),
            preferred_element_type=jnp.float32)
        h = (gp * jax.nn.sigmoid(gp) * up * wsrow).astype(jnp.bfloat16)
        yw = jax.lax.dot_general(
            w2, h, (((1,), (0,)), ((), ())),
            preferred_element_type=jnp.float32).astype(jnp.bfloat16)
        sc = jax.lax.dot_general(
            yw, gt, (((1,), (1,)), ((), ())),
            preferred_element_type=jnp.float32)
        out_ref[...] = (out_ref[...].astype(jnp.float32) +
                        sc).astype(jnp.bfloat16)


@jax.jit
def kernel(hidden_states, gate_w, w1, w3, w2):
    f32 = jnp.float32
    router_out = pl.pallas_call(
        _router_kernel,
        out_shape=[
            jax.ShapeDtypeStruct((_T, 2), jnp.int32),
            jax.ShapeDtypeStruct((_T, 2), f32),
            jax.ShapeDtypeStruct((_NB, 1), jnp.int32),
            jax.ShapeDtypeStruct((_NB, 1), jnp.int32),
            jax.ShapeDtypeStruct((_T, _H), jnp.bfloat16),
        ],
    )(hidden_states, gate_w)
    pos, tw, be, bv, x_bf = router_out
    be = be.reshape(_NB)
    bv = bv.reshape(_NB)

    bf16 = jnp.bfloat16
    i2 = _I // 2
    grid_spec = pltpu.PrefetchScalarGridSpec(
        num_scalar_prefetch=2,
        grid=(2, _NB),
        in_specs=[
            pl.BlockSpec((_T, 2), lambda i, b, be, bv: (0, 0)),
            pl.BlockSpec((_T, 2), lambda i, b, be, bv: (0, 0)),
            pl.BlockSpec((_T, _H), lambda i, b, be, bv: (0, 0)),
            pl.BlockSpec((1, i2, _H), lambda i, b, be, bv: (be[b], i, 0)),
            pl.BlockSpec((1, i2, _H), lambda i, b, be, bv: (be[b], i, 0)),
            pl.BlockSpec((1, _H, i2), lambda i, b, be, bv: (be[b], 0, i)),
        ],
        out_specs=pl.BlockSpec((_H, _T), lambda i, b, be, bv: (0, 0)),
        scratch_shapes=[
            pltpu.VMEM((_H, _NB * _B), bf16),
        ],
    )
    out = pl.pallas_call(
        _ffn_kernel,
        grid_spec=grid_spec,
        out_shape=jax.ShapeDtypeStruct((_H, _T), bf16),
        compiler_params=pltpu.CompilerParams(
            dimension_semantics=("arbitrary", "arbitrary"),
            vmem_limit_bytes=64 * 1024 * 1024),
    )(be, bv, pos, tw, x_bf, w1, w3, w2)
    return out.T.astype(f32)
